# baseline (device time: 192706 ns/iter reference)
import numpy as np
import jax
import jax.numpy as jnp
from jax import lax
from jax.experimental import pallas as pl
from jax.experimental.pallas import tpu as pltpu

N_DEV = 16
SQ = 1024
D = 1024
HQ = 8
DH = 128
HD = HQ * DH
CHUNK = SQ // N_DEV
SCALE = 0.08838834764831843


def _rope_tables():
    inv = 1.0 / (10000.0 ** (np.arange(0, DH, 2) / DH))
    pos = np.arange(SQ)[:, None] * inv[None, :]
    cos = np.repeat(np.cos(pos), 2, axis=-1).astype(np.float32)
    sin = np.repeat(np.sin(pos), 2, axis=-1).astype(np.float32)
    return np.tile(cos, (1, HQ)), np.tile(sin, (1, HQ))


_COS, _SIN = _rope_tables()


def kernel(x, Wq, Wk, Wv, Wo):
    x2 = x.reshape(SQ, D)
    cos = jnp.asarray(_COS)
    sin = jnp.asarray(_SIN)

    def body(x_ref, wq_ref, wk_ref, wv_ref, wo_ref, cos_ref, sin_ref,
             out_ref, ctx_ref, part_ref, comm_ref,
             rs_send_sems, rs_recv_sems, ag_send_sems, ag_recv_sems):
        my = lax.axis_index("i")
        right = lax.rem(my + 1, N_DEV)
        left = lax.rem(my + N_DEV - 1, N_DEV)

        barrier = pltpu.get_barrier_semaphore()
        for nbr in (left, right):
            pl.semaphore_signal(barrier, inc=1, device_id=(nbr,),
                                device_id_type=pl.DeviceIdType.MESH)
        pl.semaphore_wait(barrier, 2)

        xv = x_ref[:, :]
        cosv = cos_ref[:, :]
        sinv = sin_ref[:, :]
        col = lax.broadcasted_iota(jnp.int32, (SQ, HD), 1)
        even = (col % 2) == 0

        def rope(t):
            t_next = pltpu.roll(t, HD - 1, 1)
            t_prev = pltpu.roll(t, 1, 1)
            t_r = jnp.where(even, -t_next, t_prev)
            return t * cosv + t_r * sinv

        q = rope(jnp.dot(xv, wq_ref[:, :], preferred_element_type=jnp.float32))
        k = rope(jnp.dot(xv, wk_ref[:, :], preferred_element_type=jnp.float32))
        v = jnp.dot(xv, wv_ref[:, :], preferred_element_type=jnp.float32)

        for h in range(HQ):
            sl = slice(h * DH, (h + 1) * DH)
            s = lax.dot_general(
                q[:, sl], k[:, sl], (((1,), (1,)), ((), ())),
                preferred_element_type=jnp.float32) * SCALE
            m = jnp.max(s, axis=1, keepdims=True)
            w = jnp.exp(s - m)
            w = w / jnp.sum(w, axis=1, keepdims=True)
            ctx_ref[:, sl] = jnp.dot(w, v[:, sl],
                                     preferred_element_type=jnp.float32)

        part_ref[:, :] = jnp.dot(ctx_ref[:, :], wo_ref[:, :],
                                 preferred_element_type=jnp.float32)

        for h in range(N_DEV - 1):
            c_send = lax.rem(my - h + 2 * N_DEV, N_DEV)
            if h == 0:
                src = part_ref.at[pl.ds(c_send * CHUNK, CHUNK), :]
            else:
                src = comm_ref.at[h - 1]
            rdma = pltpu.make_async_remote_copy(
                src_ref=src,
                dst_ref=comm_ref.at[h],
                send_sem=rs_send_sems.at[h],
                recv_sem=rs_recv_sems.at[h],
                device_id=(right,),
                device_id_type=pl.DeviceIdType.MESH,
            )
            rdma.start()
            rdma.wait()
            c_recv = lax.rem(my - h - 1 + 2 * N_DEV, N_DEV)
            comm_ref[h] = (comm_ref[h]
                           + part_ref[pl.ds(c_recv * CHUNK, CHUNK), :])

        own = lax.rem(my + 1, N_DEV)
        out_ref[pl.ds(own * CHUNK, CHUNK), :] = comm_ref[N_DEV - 2]

        for g in range(N_DEV - 1):
            c_g = lax.rem(my + 1 - g + 2 * N_DEV, N_DEV)
            rdma = pltpu.make_async_remote_copy(
                src_ref=out_ref.at[pl.ds(c_g * CHUNK, CHUNK), :],
                dst_ref=out_ref.at[pl.ds(c_g * CHUNK, CHUNK), :],
                send_sem=ag_send_sems.at[g],
                recv_sem=ag_recv_sems.at[g],
                device_id=(right,),
                device_id_type=pl.DeviceIdType.MESH,
            )
            rdma.start()
            rdma.wait()

    out = pl.pallas_call(
        body,
        out_shape=jax.ShapeDtypeStruct((SQ, D), jnp.float32),
        in_specs=[pl.BlockSpec(memory_space=pltpu.VMEM)] * 7,
        out_specs=pl.BlockSpec(memory_space=pltpu.VMEM),
        scratch_shapes=[
            pltpu.VMEM((SQ, HD), jnp.float32),
            pltpu.VMEM((SQ, D), jnp.float32),
            pltpu.VMEM((N_DEV - 1, CHUNK, D), jnp.float32),
            pltpu.SemaphoreType.DMA((N_DEV - 1,)),
            pltpu.SemaphoreType.DMA((N_DEV - 1,)),
            pltpu.SemaphoreType.DMA((N_DEV - 1,)),
            pltpu.SemaphoreType.DMA((N_DEV - 1,)),
        ],
        compiler_params=pltpu.CompilerParams(
            collective_id=0,
            vmem_limit_bytes=128 * 1024 * 1024,
        ),
    )(x2, Wq, Wk, Wv, Wo, cos, sin)
    return out.reshape(1, SQ, D)


# device time: 140114 ns/iter; 1.3754x vs baseline; 1.3754x over previous
import numpy as np
import jax
import jax.numpy as jnp
from jax import lax
from jax.experimental import pallas as pl
from jax.experimental.pallas import tpu as pltpu

N_DEV = 16
SQ = 1024
D = 1024
HQ = 8
DH = 128
HD = HQ * DH
HALF = SQ // 2
SCALE = 0.08838834764831843
SIZES = (512, 256, 128, 64)


def _rope_tables():
    inv = 1.0 / (10000.0 ** (np.arange(0, DH, 2) / DH))
    pos = np.arange(SQ)[:, None] * inv[None, :]
    cos = np.repeat(np.cos(pos), 2, axis=-1).astype(np.float32)
    sin = np.repeat(np.sin(pos), 2, axis=-1).astype(np.float32)
    return np.tile(cos, (1, HQ)), np.tile(sin, (1, HQ))


_COS, _SIN = _rope_tables()


def kernel(x, Wq, Wk, Wv, Wo):
    x2 = x.reshape(SQ, D)
    cos = jnp.asarray(_COS)
    sin = jnp.asarray(_SIN)

    def body(x_ref, wq_ref, wk_ref, wv_ref, wo_ref, cos_ref, sin_ref,
             out_ref, q_ref, k_ref, v_ref, ctx_ref, part_ref,
             rbuf1, rbuf2, rbuf3, rbuf4,
             rs_send, rs_recv, ag_send, ag_recv):
        my = lax.axis_index("i")
        z = my // 4
        r = my % 4
        y = jnp.where(r >= 2, 1, 0)
        xc = jnp.where((r == 1) | (r == 2), 1, 0)
        z0 = z % 2
        z1 = z // 2

        def lid(xx, yy, zz):
            return 4 * zz + 3 * yy + xx * (1 - 2 * yy)

        partners = [
            lid(xc, 1 - y, z),
            lid(1 - xc, y, z),
            lid(xc, y, z + 1 - 2 * z0),
            lid(xc, y, z + 2 - 4 * z1),
        ]
        keep = [y * 512]
        send = [(1 - y) * 512]
        keep.append(keep[0] + xc * 256)
        send.append(keep[0] + (1 - xc) * 256)
        keep.append(keep[1] + z0 * 128)
        send.append(keep[1] + (1 - z0) * 128)
        keep.append(keep[2] + z1 * 64)
        send.append(keep[2] + (1 - z1) * 64)

        barrier = pltpu.get_barrier_semaphore()
        for p in partners:
            pl.semaphore_signal(barrier, inc=1, device_id=(p,),
                                device_id_type=pl.DeviceIdType.MESH)
        pl.semaphore_wait(barrier, 4)

        xv = x_ref[:, :]
        cosv = cos_ref[:, :]
        sinv = sin_ref[:, :]
        col = lax.broadcasted_iota(jnp.int32, (SQ, HD), 1)
        even = (col % 2) == 0

        def rope(t):
            t_next = pltpu.roll(t, HD - 1, 1)
            t_prev = pltpu.roll(t, 1, 1)
            t_r = jnp.where(even, -t_next, t_prev)
            return t * cosv + t_r * sinv

        q_ref[:, :] = rope(jnp.dot(xv, wq_ref[:, :],
                                   preferred_element_type=jnp.float32))
        k_ref[:, :] = rope(jnp.dot(xv, wk_ref[:, :],
                                   preferred_element_type=jnp.float32))
        v_ref[:, :] = jnp.dot(xv, wv_ref[:, :],
                              preferred_element_type=jnp.float32)

        def attn_part(base):
            for h in range(HQ):
                sl = slice(h * DH, (h + 1) * DH)
                s = lax.dot_general(
                    q_ref[pl.ds(base, HALF), sl], k_ref[:, sl],
                    (((1,), (1,)), ((), ())),
                    preferred_element_type=jnp.float32) * SCALE
                m = jnp.max(s, axis=1, keepdims=True)
                w = jnp.exp(s - m)
                w = w / jnp.sum(w, axis=1, keepdims=True)
                ctx_ref[pl.ds(base, HALF), sl] = jnp.dot(
                    w, v_ref[:, sl], preferred_element_type=jnp.float32)
            part_ref[pl.ds(base, HALF), :] = jnp.dot(
                ctx_ref[pl.ds(base, HALF), :], wo_ref[:, :],
                preferred_element_type=jnp.float32)

        rbufs = [rbuf1, rbuf2, rbuf3, rbuf4]

        def rs_step(s):
            return pltpu.make_async_remote_copy(
                src_ref=part_ref.at[pl.ds(send[s], SIZES[s]), :],
                dst_ref=rbufs[s],
                send_sem=rs_send.at[s],
                recv_sem=rs_recv.at[s],
                device_id=(partners[s],),
                device_id_type=pl.DeviceIdType.MESH,
            )

        attn_part(send[0])
        rdma1 = rs_step(0)
        rdma1.start()
        attn_part(keep[0])
        rdma1.wait()
        part_ref[pl.ds(keep[0], SIZES[0]), :] = (
            part_ref[pl.ds(keep[0], SIZES[0]), :] + rbuf1[:, :])

        for s in (1, 2, 3):
            rdma = rs_step(s)
            rdma.start()
            rdma.wait()
            part_ref[pl.ds(keep[s], SIZES[s]), :] = (
                part_ref[pl.ds(keep[s], SIZES[s]), :] + rbufs[s][:, :])

        out_ref[pl.ds(keep[3], 64), :] = part_ref[pl.ds(keep[3], 64), :]

        for s in (3, 2, 1, 0):
            rdma = pltpu.make_async_remote_copy(
                src_ref=out_ref.at[pl.ds(keep[s], SIZES[s]), :],
                dst_ref=out_ref.at[pl.ds(keep[s], SIZES[s]), :],
                send_sem=ag_send.at[s],
                recv_sem=ag_recv.at[s],
                device_id=(partners[s],),
                device_id_type=pl.DeviceIdType.MESH,
            )
            rdma.start()
            rdma.wait()

    out = pl.pallas_call(
        body,
        out_shape=jax.ShapeDtypeStruct((SQ, D), jnp.float32),
        in_specs=[pl.BlockSpec(memory_space=pltpu.VMEM)] * 7,
        out_specs=pl.BlockSpec(memory_space=pltpu.VMEM),
        scratch_shapes=[
            pltpu.VMEM((SQ, HD), jnp.float32),
            pltpu.VMEM((SQ, HD), jnp.float32),
            pltpu.VMEM((SQ, HD), jnp.float32),
            pltpu.VMEM((SQ, HD), jnp.float32),
            pltpu.VMEM((SQ, D), jnp.float32),
            pltpu.VMEM((512, D), jnp.float32),
            pltpu.VMEM((256, D), jnp.float32),
            pltpu.VMEM((128, D), jnp.float32),
            pltpu.VMEM((64, D), jnp.float32),
            pltpu.SemaphoreType.DMA((4,)),
            pltpu.SemaphoreType.DMA((4,)),
            pltpu.SemaphoreType.DMA((4,)),
            pltpu.SemaphoreType.DMA((4,)),
        ],
        compiler_params=pltpu.CompilerParams(
            collective_id=0,
            vmem_limit_bytes=128 * 1024 * 1024,
        ),
    )(x2, Wq, Wk, Wv, Wo, cos, sin)
    return out.reshape(1, SQ, D)


# device time: 92398 ns/iter; 2.0856x vs baseline; 1.5164x over previous
import numpy as np
import jax
import jax.numpy as jnp
from jax import lax
from jax.experimental import pallas as pl
from jax.experimental.pallas import tpu as pltpu

N_DEV = 16
SQ = 1024
D = 1024
HQ = 8
DH = 128
HD = HQ * DH
HALF = SQ // 2
SCALE = 0.08838834764831843
SIZES = (512, 256, 128, 64)


def _rope_tables():
    inv = 1.0 / (10000.0 ** (np.arange(0, DH, 2) / DH))
    pos = np.arange(SQ)[:, None] * inv[None, :]
    cos = np.repeat(np.cos(pos), 2, axis=-1).astype(np.float32)
    sin = np.repeat(np.sin(pos), 2, axis=-1).astype(np.float32)
    return cos, sin


_COS, _SIN = _rope_tables()


def kernel(x, Wq, Wk, Wv, Wo):
    x2 = x.reshape(SQ, D)
    cos = jnp.asarray(_COS)
    sin = jnp.asarray(_SIN)

    def body(x_ref, wq_ref, wk_ref, wv_ref, wo_ref, cos_ref, sin_ref,
             out_ref, q_ref, k_ref, v_ref, ctx_ref, part_ref,
             sbuf1, sbuf2, sbuf3, sbuf4,
             rbuf1, rbuf2, rbuf3, rbuf4,
             rs_send, rs_recv, ag_send, ag_recv):
        my = lax.axis_index("i")
        z = my // 4
        r = my % 4
        y = jnp.where(r >= 2, 1, 0)
        xc = jnp.where((r == 1) | (r == 2), 1, 0)
        z0 = z % 2
        z1 = z // 2

        def lid(xx, yy, zz):
            return 4 * zz + 3 * yy + xx * (1 - 2 * yy)

        partners = [
            lid(xc, 1 - y, z),
            lid(1 - xc, y, z),
            lid(xc, y, z + 1 - 2 * z0),
            lid(xc, y, z + 2 - 4 * z1),
        ]
        keep = [y * 512]
        send = [(1 - y) * 512]
        keep.append(keep[0] + xc * 256)
        send.append(keep[0] + (1 - xc) * 256)
        keep.append(keep[1] + z0 * 128)
        send.append(keep[1] + (1 - z0) * 128)
        keep.append(keep[2] + z1 * 64)
        send.append(keep[2] + (1 - z1) * 64)

        barrier = pltpu.get_barrier_semaphore()
        for p in partners:
            pl.semaphore_signal(barrier, inc=1, device_id=(p,),
                                device_id_type=pl.DeviceIdType.MESH)
        pl.semaphore_wait(barrier, 4)

        xv = x_ref[:, :]
        cosv = jnp.concatenate([cos_ref[:, :]] * HQ, axis=1)
        sinv = jnp.concatenate([sin_ref[:, :]] * HQ, axis=1)
        col = lax.broadcasted_iota(jnp.int32, (SQ, HD), 1)
        even = (col % 2) == 0

        def rope(t):
            t_next = pltpu.roll(t, HD - 1, 1)
            t_prev = pltpu.roll(t, 1, 1)
            t_r = jnp.where(even, -t_next, t_prev)
            return t * cosv + t_r * sinv

        q_ref[:, :] = rope(jnp.dot(xv, wq_ref[:, :],
                                   preferred_element_type=jnp.float32))
        k_ref[:, :] = rope(jnp.dot(xv, wk_ref[:, :],
                                   preferred_element_type=jnp.float32))
        v_ref[:, :] = jnp.dot(xv, wv_ref[:, :],
                              preferred_element_type=jnp.float32)

        def attn_part(base):
            for h in range(HQ):
                sl = slice(h * DH, (h + 1) * DH)
                s = lax.dot_general(
                    q_ref[pl.ds(base, HALF), sl], k_ref[:, sl],
                    (((1,), (1,)), ((), ())),
                    preferred_element_type=jnp.float32) * SCALE
                m = jnp.max(s, axis=1, keepdims=True)
                w = jnp.exp(s - m)
                w = w / jnp.sum(w, axis=1, keepdims=True)
                ctx_ref[:, sl] = jnp.dot(
                    w, v_ref[:, sl], preferred_element_type=jnp.float32)
            part_ref[pl.ds(base, HALF), :] = jnp.dot(
                ctx_ref[:, :], wo_ref[:, :],
                preferred_element_type=jnp.float32)

        rbufs = [rbuf1, rbuf2, rbuf3, rbuf4]
        sbufs = [sbuf1, sbuf2, sbuf3, sbuf4]

        def rs_step(s):
            sbufs[s][:, :] = part_ref[pl.ds(send[s], SIZES[s]), :].astype(
                jnp.bfloat16)
            return pltpu.make_async_remote_copy(
                src_ref=sbufs[s],
                dst_ref=rbufs[s],
                send_sem=rs_send.at[s],
                recv_sem=rs_recv.at[s],
                device_id=(partners[s],),
                device_id_type=pl.DeviceIdType.MESH,
            )

        attn_part(send[0])
        rdma1 = rs_step(0)
        rdma1.start()
        attn_part(keep[0])
        rdma1.wait()
        part_ref[pl.ds(keep[0], SIZES[0]), :] = (
            part_ref[pl.ds(keep[0], SIZES[0]), :]
            + rbuf1[:, :].astype(jnp.float32))

        for s in (1, 2, 3):
            rdma = rs_step(s)
            rdma.start()
            rdma.wait()
            part_ref[pl.ds(keep[s], SIZES[s]), :] = (
                part_ref[pl.ds(keep[s], SIZES[s]), :]
                + rbufs[s][:, :].astype(jnp.float32))

        out_ref[pl.ds(keep[3], 64), :] = part_ref[pl.ds(keep[3], 64),
                                                  :].astype(jnp.bfloat16)

        for s in (3, 2, 1, 0):
            rdma = pltpu.make_async_remote_copy(
                src_ref=out_ref.at[pl.ds(keep[s], SIZES[s]), :],
                dst_ref=out_ref.at[pl.ds(keep[s], SIZES[s]), :],
                send_sem=ag_send.at[s],
                recv_sem=ag_recv.at[s],
                device_id=(partners[s],),
                device_id_type=pl.DeviceIdType.MESH,
            )
            rdma.start()
            rdma.wait()

    out = pl.pallas_call(
        body,
        out_shape=jax.ShapeDtypeStruct((SQ, D), jnp.bfloat16),
        in_specs=[pl.BlockSpec(memory_space=pltpu.VMEM)] * 7,
        out_specs=pl.BlockSpec(memory_space=pltpu.VMEM),
        scratch_shapes=[
            pltpu.VMEM((SQ, HD), jnp.float32),
            pltpu.VMEM((SQ, HD), jnp.float32),
            pltpu.VMEM((SQ, HD), jnp.float32),
            pltpu.VMEM((HALF, HD), jnp.float32),
            pltpu.VMEM((SQ, D), jnp.float32),
            pltpu.VMEM((512, D), jnp.bfloat16),
            pltpu.VMEM((256, D), jnp.bfloat16),
            pltpu.VMEM((128, D), jnp.bfloat16),
            pltpu.VMEM((64, D), jnp.bfloat16),
            pltpu.VMEM((512, D), jnp.bfloat16),
            pltpu.VMEM((256, D), jnp.bfloat16),
            pltpu.VMEM((128, D), jnp.bfloat16),
            pltpu.VMEM((64, D), jnp.bfloat16),
            pltpu.SemaphoreType.DMA((4,)),
            pltpu.SemaphoreType.DMA((4,)),
            pltpu.SemaphoreType.DMA((4,)),
            pltpu.SemaphoreType.DMA((4,)),
        ],
        compiler_params=pltpu.CompilerParams(
            collective_id=0,
            vmem_limit_bytes=128 * 1024 * 1024,
        ),
    )(x2, Wq, Wk, Wv, Wo, cos, sin)
    return out.astype(jnp.float32).reshape(1, SQ, D)
